# baseline (speedup 1.0000x reference)
import functools

import jax
import jax.numpy as jnp
from jax.experimental import pallas as pl
from jax.experimental.pallas import tpu as pltpu


def _gap_kernel(x_ref, o_ref, *, inv_hw):
    acc = jnp.sum(x_ref[...].astype(jnp.float32), axis=0)
    acc = jnp.sum(acc, axis=0)
    o_ref[...] = (acc * inv_hw).astype(o_ref.dtype)


def kernel(x):
    N, C, H, W = x.shape
    xt = jnp.transpose(x, (2, 3, 0, 1))

    bn = N
    for cand in (8, 4, 2, 1):
        if N % cand == 0:
            bn = cand
            break
    n_tiles = N // bn

    out = pl.pallas_call(
        functools.partial(_gap_kernel, inv_hw=1.0 / float(H * W)),
        out_shape=jax.ShapeDtypeStruct((N, C), x.dtype),
        grid=(n_tiles,),
        in_specs=[pl.BlockSpec((H, W, bn, C), lambda i: (0, 0, i, 0))],
        out_specs=pl.BlockSpec((bn, C), lambda i: (i, 0)),
        compiler_params=pltpu.CompilerParams(
            dimension_semantics=("parallel",),
            vmem_limit_bytes=64 * 1024 * 1024,
        ),
    )(xt)
    return out


# speedup vs baseline: 1.1565x; 1.1565x over previous
import functools

import jax
import jax.numpy as jnp
from jax.experimental import pallas as pl
from jax.experimental.pallas import tpu as pltpu


def _gap_kernel(x_ref, o_ref, *, inv_hw):
    acc = jnp.sum(x_ref[...].astype(jnp.float32), axis=0)
    acc = jnp.sum(acc, axis=0)
    o_ref[...] = (acc * inv_hw).astype(o_ref.dtype)


def kernel(x):
    N, C, H, W = x.shape
    xt = jnp.transpose(x, (2, 3, 0, 1))

    bn = N
    for cand in (16, 8, 4, 2, 1):
        if N % cand == 0:
            bn = cand
            break
    n_tiles = N // bn

    out = pl.pallas_call(
        functools.partial(_gap_kernel, inv_hw=1.0 / float(H * W)),
        out_shape=jax.ShapeDtypeStruct((N, C), x.dtype),
        grid=(n_tiles,),
        in_specs=[pl.BlockSpec((H, W, bn, C), lambda i: (0, 0, i, 0))],
        out_specs=pl.BlockSpec((bn, C), lambda i: (i, 0)),
        compiler_params=pltpu.CompilerParams(
            dimension_semantics=("parallel",),
            vmem_limit_bytes=64 * 1024 * 1024,
        ),
    )(xt)
    return out
